# baseline (speedup 1.0000x reference)
import functools

import jax
import jax.numpy as jnp
from jax import lax
from jax.experimental import pallas as pl
from jax.experimental.pallas import tpu as pltpu
from jax.experimental.pallas import tpu_sc as plsc

_B = 16384
_D = 64
_VD = 512
_NI = 100000
_NU = 1000000
_L2_LAMBDA = 1e-05

_ROWS_PER_BLK = 1000
_CHUNK = 128
_NW = 32
_PER_W = _B // _NW
_LOSS_BLK = 2048


def _fuse_items_body(ie_ref, ivf_ref, wt_ref, out_ref):
    out_ref[:, :_D] = ie_ref[...]
    out_ref[:, _D:] = jnp.dot(ivf_ref[...], wt_ref[...],
                              preferred_element_type=jnp.float32)


def _fuse_items(ie, ivf, w_t):
    grid = _NI // _ROWS_PER_BLK
    return pl.pallas_call(
        _fuse_items_body,
        grid=(grid,),
        in_specs=[
            pl.BlockSpec((_ROWS_PER_BLK, _D), lambda i: (i, 0)),
            pl.BlockSpec((_ROWS_PER_BLK, _VD), lambda i: (i, 0)),
            pl.BlockSpec((_VD, _D), lambda i: (0, 0)),
        ],
        out_specs=pl.BlockSpec((_ROWS_PER_BLK, 2 * _D), lambda i: (i, 0)),
        out_shape=jax.ShapeDtypeStruct((_NI, 2 * _D), jnp.float32),
    )(ie, ivf, w_t)


def _user_kernel_body(uid, ue2, uv2, o_ue, o_uv, idx_v, rows_v, sem):
    wid = lax.axis_index("s") * 2 + lax.axis_index("c")
    base0 = wid * _PER_W

    def gather_one(table, out, base):
        pltpu.async_copy(table.at[idx_v], rows_v, sem).wait()
        pltpu.sync_copy(rows_v, out.at[pl.ds(base, _CHUNK)])

    for c in range(_PER_W // _CHUNK):
        base = base0 + c * _CHUNK
        pltpu.sync_copy(uid.at[pl.ds(base, _CHUNK)], idx_v)
        gather_one(ue2, o_ue, base)
        gather_one(uv2, o_uv, base)


def _gather_users(uid, ue2, uv2):
    mesh = plsc.VectorSubcoreMesh(core_axis_name="c", subcore_axis_name="s",
                                  num_cores=2, num_subcores=16)
    urows = jax.ShapeDtypeStruct((_B, _D), jnp.float32)
    k = pl.kernel(
        _user_kernel_body,
        out_type=(urows, urows),
        mesh=mesh,
        scratch_types=[
            pltpu.VMEM((_CHUNK,), jnp.int32),
            pltpu.VMEM((_CHUNK, _D), jnp.float32),
            pltpu.SemaphoreType.DMA,
        ],
        compiler_params=pltpu.CompilerParams(use_tc_tiling_on_sc=False),
    )
    return k(uid, ue2, uv2)


def _item_kernel_body(pid, nid, it_t, o_it_p, o_it_n, idx_v, rows_v, semi):
    wid = lax.axis_index("s") * 2 + lax.axis_index("c")
    base0 = wid * _PER_W

    for c in range(_PER_W // _CHUNK):
        base = base0 + c * _CHUNK
        pltpu.sync_copy(pid.at[pl.ds(base, _CHUNK)], idx_v)
        pltpu.async_copy(it_t.at[idx_v], rows_v, semi).wait()
        pltpu.sync_copy(rows_v, o_it_p.at[pl.ds(base, _CHUNK)])
        pltpu.sync_copy(nid.at[pl.ds(base, _CHUNK)], idx_v)
        pltpu.async_copy(it_t.at[idx_v], rows_v, semi).wait()
        pltpu.sync_copy(rows_v, o_it_n.at[pl.ds(base, _CHUNK)])


def _gather_items(pid, nid, it_t):
    mesh = plsc.VectorSubcoreMesh(core_axis_name="c", subcore_axis_name="s",
                                  num_cores=2, num_subcores=16)
    irows = jax.ShapeDtypeStruct((_B, 2 * _D), jnp.float32)
    k = pl.kernel(
        _item_kernel_body,
        out_type=(irows, irows),
        mesh=mesh,
        scratch_types=[
            pltpu.VMEM((_CHUNK,), jnp.int32),
            pltpu.VMEM((_CHUNK, 2 * _D), jnp.float32),
            pltpu.SemaphoreType.DMA,
        ],
    )
    return k(pid, nid, it_t)


def _loss_body(ue, uv, itp, itn, out_ref):
    i = pl.program_id(0)
    ue_ = ue[...]
    uv_ = uv[...]
    ucomb = jnp.concatenate([ue_, uv_], axis=1)
    itp_ = itp[...]
    itn_ = itn[...]
    pos = jnp.sum(ucomb * itp_, axis=1)
    neg = jnp.sum(ucomb * itn_, axis=1)
    x = pos - neg
    cf = jnp.maximum(-x, 0.0) + jnp.log1p(jnp.exp(-jnp.abs(x)))
    l2 = 0.5 * (jnp.sum(ue_ * ue_, axis=1)
                + jnp.sum(itp_[:, :_D] * itp_[:, :_D], axis=1)
                + jnp.sum(itn_[:, :_D] * itn_[:, :_D], axis=1))
    part = jnp.sum(cf + _L2_LAMBDA * l2)

    @pl.when(i == 0)
    def _():
        out_ref[0, 0] = 0.0

    out_ref[0, 0] += part

    @pl.when(i == pl.num_programs(0) - 1)
    def _():
        out_ref[0, 0] = out_ref[0, 0] / float(_B)


def _fused_loss(ue, uv, itp, itn):
    grid = _B // _LOSS_BLK
    ublk = pl.BlockSpec((_LOSS_BLK, _D), lambda i: (i, 0))
    iblk = pl.BlockSpec((_LOSS_BLK, 2 * _D), lambda i: (i, 0))
    return pl.pallas_call(
        _loss_body,
        grid=(grid,),
        in_specs=[ublk, ublk, iblk, iblk],
        out_specs=pl.BlockSpec((1, 1), lambda i: (0, 0),
                               memory_space=pltpu.SMEM),
        out_shape=jax.ShapeDtypeStruct((1, 1), jnp.float32),
    )(ue, uv, itp, itn)


def kernel(user_ids, item_pos_ids, item_neg_ids, user_embed, item_embed,
           user_visual_embed, item_visual_feature, W_vis):
    ue, uv = _gather_users(user_ids, user_embed, user_visual_embed)
    fused_items = _fuse_items(item_embed, item_visual_feature, W_vis.T)
    itp, itn = _gather_items(item_pos_ids, item_neg_ids, fused_items)
    loss = _fused_loss(ue, uv, itp, itn)
    return loss[0, 0]


# speedup vs baseline: 1.4023x; 1.4023x over previous
import functools

import jax
import jax.numpy as jnp
from jax import lax
from jax.experimental import pallas as pl
from jax.experimental.pallas import tpu as pltpu
from jax.experimental.pallas import tpu_sc as plsc

_B = 16384
_D = 64
_VD = 512
_NI = 100000
_NU = 1000000
_L2_LAMBDA = 1e-05

_ROWS_PER_BLK = 1000
_CHUNK = 128
_NW = 32
_PER_W = _B // _NW
_LOSS_BLK = 2048


def _fuse_items_body(ie_ref, ivf_ref, wt_ref, out_ref):
    out_ref[:, :_D] = ie_ref[...]
    out_ref[:, _D:] = jnp.dot(ivf_ref[...], wt_ref[...],
                              preferred_element_type=jnp.float32)


def _fuse_items(ie, ivf, w_t):
    grid = _NI // _ROWS_PER_BLK
    return pl.pallas_call(
        _fuse_items_body,
        grid=(grid,),
        in_specs=[
            pl.BlockSpec((_ROWS_PER_BLK, _D), lambda i: (i, 0)),
            pl.BlockSpec((_ROWS_PER_BLK, _VD), lambda i: (i, 0)),
            pl.BlockSpec((_VD, _D), lambda i: (0, 0)),
        ],
        out_specs=pl.BlockSpec((_ROWS_PER_BLK, 2 * _D), lambda i: (i, 0)),
        out_shape=jax.ShapeDtypeStruct((_NI, 2 * _D), jnp.float32),
    )(ie, ivf, w_t)


def _gather_kernel_body(uid, pid, nid, ue2, uv2, it_t,
                        o_ue, o_uv, o_it_p, o_it_n,
                        uidv, idx_v, rows_v, uerows, uvrows, sem, semi):
    wid = lax.axis_index("s") * 2 + lax.axis_index("c")
    base0 = wid * _PER_W

    for c in range(_PER_W // _CHUNK):
        base = base0 + c * _CHUNK

        pltpu.sync_copy(uid.at[pl.ds(base, _CHUNK)], uidv)

        @pl.loop(0, _CHUNK // 16)
        def _(g):
            vec = uidv[pl.ds(g * 16, 16)]
            descs = []
            for j in range(16):
                e = g * 16 + j
                rid = vec[j]
                descs.append(pltpu.async_copy(
                    ue2.at[pl.ds(rid, 1)], uerows.at[pl.ds(e, 1)], sem))
                descs.append(pltpu.async_copy(
                    uv2.at[pl.ds(rid, 1)], uvrows.at[pl.ds(e, 1)], sem))
            for dsc in descs:
                dsc.wait()

        pltpu.sync_copy(uerows, o_ue.at[pl.ds(base, _CHUNK)])
        pltpu.sync_copy(uvrows, o_uv.at[pl.ds(base, _CHUNK)])

        pltpu.sync_copy(pid.at[pl.ds(base, _CHUNK)], idx_v)
        pltpu.async_copy(it_t.at[idx_v], rows_v, semi).wait()
        pltpu.sync_copy(rows_v, o_it_p.at[pl.ds(base, _CHUNK)])
        pltpu.sync_copy(nid.at[pl.ds(base, _CHUNK)], idx_v)
        pltpu.async_copy(it_t.at[idx_v], rows_v, semi).wait()
        pltpu.sync_copy(rows_v, o_it_n.at[pl.ds(base, _CHUNK)])


def _gather_all(uid, pid, nid, ue2, uv2, it_t):
    mesh = plsc.VectorSubcoreMesh(core_axis_name="c", subcore_axis_name="s",
                                  num_cores=2, num_subcores=16)
    urows = jax.ShapeDtypeStruct((_B, _D), jnp.float32)
    irows = jax.ShapeDtypeStruct((_B, 2 * _D), jnp.float32)
    k = pl.kernel(
        _gather_kernel_body,
        out_type=(urows, urows, irows, irows),
        mesh=mesh,
        scratch_types=[
            pltpu.VMEM((_CHUNK,), jnp.int32),
            pltpu.VMEM((_CHUNK,), jnp.int32),
            pltpu.VMEM((_CHUNK, 2 * _D), jnp.float32),
            pltpu.VMEM((_CHUNK, _D), jnp.float32),
            pltpu.VMEM((_CHUNK, _D), jnp.float32),
            pltpu.SemaphoreType.DMA,
            pltpu.SemaphoreType.DMA,
        ],
    )
    return k(uid, pid, nid, ue2, uv2, it_t)


def _loss_body(ue, uv, itp, itn, out_ref):
    i = pl.program_id(0)
    ue_ = ue[...]
    uv_ = uv[...]
    ucomb = jnp.concatenate([ue_, uv_], axis=1)
    itp_ = itp[...]
    itn_ = itn[...]
    pos = jnp.sum(ucomb * itp_, axis=1)
    neg = jnp.sum(ucomb * itn_, axis=1)
    x = pos - neg
    cf = jnp.maximum(-x, 0.0) + jnp.log1p(jnp.exp(-jnp.abs(x)))
    l2 = 0.5 * (jnp.sum(ue_ * ue_, axis=1)
                + jnp.sum(itp_[:, :_D] * itp_[:, :_D], axis=1)
                + jnp.sum(itn_[:, :_D] * itn_[:, :_D], axis=1))
    part = jnp.sum(cf + _L2_LAMBDA * l2)

    @pl.when(i == 0)
    def _():
        out_ref[0, 0] = 0.0

    out_ref[0, 0] += part

    @pl.when(i == pl.num_programs(0) - 1)
    def _():
        out_ref[0, 0] = out_ref[0, 0] / float(_B)


def _fused_loss(ue, uv, itp, itn):
    grid = _B // _LOSS_BLK
    ublk = pl.BlockSpec((_LOSS_BLK, _D), lambda i: (i, 0))
    iblk = pl.BlockSpec((_LOSS_BLK, 2 * _D), lambda i: (i, 0))
    return pl.pallas_call(
        _loss_body,
        grid=(grid,),
        in_specs=[ublk, ublk, iblk, iblk],
        out_specs=pl.BlockSpec((1, 1), lambda i: (0, 0),
                               memory_space=pltpu.SMEM),
        out_shape=jax.ShapeDtypeStruct((1, 1), jnp.float32),
    )(ue, uv, itp, itn)


def kernel(user_ids, item_pos_ids, item_neg_ids, user_embed, item_embed,
           user_visual_embed, item_visual_feature, W_vis):
    fused_items = _fuse_items(item_embed, item_visual_feature, W_vis.T)
    ue, uv, itp, itn = _gather_all(
        user_ids, item_pos_ids, item_neg_ids,
        user_embed, user_visual_embed, fused_items)
    loss = _fused_loss(ue, uv, itp, itn)
    return loss[0, 0]


# speedup vs baseline: 1.4540x; 1.0369x over previous
import functools

import jax
import jax.numpy as jnp
from jax import lax
from jax.experimental import pallas as pl
from jax.experimental.pallas import tpu as pltpu
from jax.experimental.pallas import tpu_sc as plsc

_B = 16384
_D = 64
_VD = 512
_NI = 100000
_NU = 1000000
_L2_LAMBDA = 1e-05

_ROWS_PER_BLK = 2000
_CHUNK = 128
_NW = 32
_PER_W = _B // _NW
_LOSS_BLK = 2048


def _fuse_items_body(ie_ref, ivf_ref, wt_ref, out_ref):
    out_ref[:, :_D] = ie_ref[...]
    out_ref[:, _D:] = jnp.dot(ivf_ref[...], wt_ref[...],
                              preferred_element_type=jnp.float32)


def _fuse_items(ie, ivf, w_t):
    grid = _NI // _ROWS_PER_BLK
    return pl.pallas_call(
        _fuse_items_body,
        grid=(grid,),
        in_specs=[
            pl.BlockSpec((_ROWS_PER_BLK, _D), lambda i: (i, 0)),
            pl.BlockSpec((_ROWS_PER_BLK, _VD), lambda i: (i, 0)),
            pl.BlockSpec((_VD, _D), lambda i: (0, 0)),
        ],
        out_specs=pl.BlockSpec((_ROWS_PER_BLK, 2 * _D), lambda i: (i, 0)),
        out_shape=jax.ShapeDtypeStruct((_NI, 2 * _D), jnp.float32),
    )(ie, ivf, w_t)


def _gather_kernel_body(uid, pid, nid, ue2, uv2, it_t,
                        o_ue, o_uv, o_it_p, o_it_n,
                        uidv, idx_v, rows_v, uerows, uvrows, sem, semi):
    wid = lax.axis_index("s") * 2 + lax.axis_index("c")
    base0 = wid * _PER_W

    for c in range(_PER_W // _CHUNK):
        base = base0 + c * _CHUNK

        pltpu.sync_copy(uid.at[pl.ds(base, _CHUNK)], uidv)

        @pl.loop(0, _CHUNK // 16)
        def _(g):
            vec = uidv[pl.ds(g * 16, 16)]
            descs = []
            for j in range(16):
                e = g * 16 + j
                rid = vec[j]
                descs.append(pltpu.async_copy(
                    ue2.at[pl.ds(rid, 1)], uerows.at[pl.ds(e, 1)], sem))
                descs.append(pltpu.async_copy(
                    uv2.at[pl.ds(rid, 1)], uvrows.at[pl.ds(e, 1)], sem))
            for dsc in descs:
                dsc.wait()

        pltpu.sync_copy(uerows, o_ue.at[pl.ds(base, _CHUNK)])
        pltpu.sync_copy(uvrows, o_uv.at[pl.ds(base, _CHUNK)])

        pltpu.sync_copy(pid.at[pl.ds(base, _CHUNK)], idx_v)
        pltpu.async_copy(it_t.at[idx_v], rows_v, semi).wait()
        pltpu.sync_copy(rows_v, o_it_p.at[pl.ds(base, _CHUNK)])
        pltpu.sync_copy(nid.at[pl.ds(base, _CHUNK)], idx_v)
        pltpu.async_copy(it_t.at[idx_v], rows_v, semi).wait()
        pltpu.sync_copy(rows_v, o_it_n.at[pl.ds(base, _CHUNK)])


def _gather_all(uid, pid, nid, ue2, uv2, it_t):
    mesh = plsc.VectorSubcoreMesh(core_axis_name="c", subcore_axis_name="s",
                                  num_cores=2, num_subcores=16)
    urows = jax.ShapeDtypeStruct((_B, _D), jnp.float32)
    irows = jax.ShapeDtypeStruct((_B, 2 * _D), jnp.float32)
    k = pl.kernel(
        _gather_kernel_body,
        out_type=(urows, urows, irows, irows),
        mesh=mesh,
        scratch_types=[
            pltpu.VMEM((_CHUNK,), jnp.int32),
            pltpu.VMEM((_CHUNK,), jnp.int32),
            pltpu.VMEM((_CHUNK, 2 * _D), jnp.float32),
            pltpu.VMEM((_CHUNK, _D), jnp.float32),
            pltpu.VMEM((_CHUNK, _D), jnp.float32),
            pltpu.SemaphoreType.DMA,
            pltpu.SemaphoreType.DMA,
        ],
    )
    return k(uid, pid, nid, ue2, uv2, it_t)


def _loss_body(ue, uv, itp, itn, out_ref):
    i = pl.program_id(0)
    ue_ = ue[...]
    uv_ = uv[...]
    ucomb = jnp.concatenate([ue_, uv_], axis=1)
    itp_ = itp[...]
    itn_ = itn[...]
    pos = jnp.sum(ucomb * itp_, axis=1)
    neg = jnp.sum(ucomb * itn_, axis=1)
    x = pos - neg
    cf = jnp.maximum(-x, 0.0) + jnp.log1p(jnp.exp(-jnp.abs(x)))
    l2 = 0.5 * (jnp.sum(ue_ * ue_, axis=1)
                + jnp.sum(itp_[:, :_D] * itp_[:, :_D], axis=1)
                + jnp.sum(itn_[:, :_D] * itn_[:, :_D], axis=1))
    part = jnp.sum(cf + _L2_LAMBDA * l2)

    @pl.when(i == 0)
    def _():
        out_ref[0, 0] = 0.0

    out_ref[0, 0] += part

    @pl.when(i == pl.num_programs(0) - 1)
    def _():
        out_ref[0, 0] = out_ref[0, 0] / float(_B)


def _fused_loss(ue, uv, itp, itn):
    grid = _B // _LOSS_BLK
    ublk = pl.BlockSpec((_LOSS_BLK, _D), lambda i: (i, 0))
    iblk = pl.BlockSpec((_LOSS_BLK, 2 * _D), lambda i: (i, 0))
    return pl.pallas_call(
        _loss_body,
        grid=(grid,),
        in_specs=[ublk, ublk, iblk, iblk],
        out_specs=pl.BlockSpec((1, 1), lambda i: (0, 0),
                               memory_space=pltpu.SMEM),
        out_shape=jax.ShapeDtypeStruct((1, 1), jnp.float32),
    )(ue, uv, itp, itn)


def kernel(user_ids, item_pos_ids, item_neg_ids, user_embed, item_embed,
           user_visual_embed, item_visual_feature, W_vis):
    fused_items = _fuse_items(item_embed, item_visual_feature, W_vis.T)
    ue, uv, itp, itn = _gather_all(
        user_ids, item_pos_ids, item_neg_ids,
        user_embed, user_visual_embed, fused_items)
    loss = _fused_loss(ue, uv, itp, itn)
    return loss[0, 0]


# speedup vs baseline: 1.4664x; 1.0085x over previous
import functools

import jax
import jax.numpy as jnp
from jax import lax
from jax.experimental import pallas as pl
from jax.experimental.pallas import tpu as pltpu
from jax.experimental.pallas import tpu_sc as plsc

_B = 16384
_D = 64
_VD = 512
_NI = 100000
_NU = 1000000
_L2_LAMBDA = 1e-05

_ROWS_PER_BLK = 4000
_CHUNK = 128
_NW = 32
_PER_W = _B // _NW
_LOSS_BLK = 2048


def _fuse_items_body(ie_ref, ivf_ref, wt_ref, out_ref):
    out_ref[:, :_D] = ie_ref[...]
    out_ref[:, _D:] = jnp.dot(ivf_ref[...], wt_ref[...],
                              preferred_element_type=jnp.float32)


def _fuse_items(ie, ivf, w_t):
    grid = _NI // _ROWS_PER_BLK
    return pl.pallas_call(
        _fuse_items_body,
        grid=(grid,),
        in_specs=[
            pl.BlockSpec((_ROWS_PER_BLK, _D), lambda i: (i, 0)),
            pl.BlockSpec((_ROWS_PER_BLK, _VD), lambda i: (i, 0)),
            pl.BlockSpec((_VD, _D), lambda i: (0, 0)),
        ],
        out_specs=pl.BlockSpec((_ROWS_PER_BLK, 2 * _D), lambda i: (i, 0)),
        out_shape=jax.ShapeDtypeStruct((_NI, 2 * _D), jnp.float32),
    )(ie, ivf, w_t)


def _gather_kernel_body(uid, pid, nid, ue2, uv2, it_t,
                        o_ue, o_uv, o_it_p, o_it_n,
                        uidv, idx_v, rows_v, uerows, uvrows, sem, semi):
    wid = lax.axis_index("s") * 2 + lax.axis_index("c")
    base0 = wid * _PER_W

    for c in range(_PER_W // _CHUNK):
        base = base0 + c * _CHUNK

        pltpu.sync_copy(uid.at[pl.ds(base, _CHUNK)], uidv)

        @pl.loop(0, _CHUNK // 16)
        def _(g):
            vec = uidv[pl.ds(g * 16, 16)]
            descs = []
            for j in range(16):
                e = g * 16 + j
                rid = vec[j]
                descs.append(pltpu.async_copy(
                    ue2.at[pl.ds(rid, 1)], uerows.at[pl.ds(e, 1)], sem))
                descs.append(pltpu.async_copy(
                    uv2.at[pl.ds(rid, 1)], uvrows.at[pl.ds(e, 1)], sem))
            for dsc in descs:
                dsc.wait()

        pltpu.sync_copy(uerows, o_ue.at[pl.ds(base, _CHUNK)])
        pltpu.sync_copy(uvrows, o_uv.at[pl.ds(base, _CHUNK)])

        pltpu.sync_copy(pid.at[pl.ds(base, _CHUNK)], idx_v)
        pltpu.async_copy(it_t.at[idx_v], rows_v, semi).wait()
        pltpu.sync_copy(rows_v, o_it_p.at[pl.ds(base, _CHUNK)])
        pltpu.sync_copy(nid.at[pl.ds(base, _CHUNK)], idx_v)
        pltpu.async_copy(it_t.at[idx_v], rows_v, semi).wait()
        pltpu.sync_copy(rows_v, o_it_n.at[pl.ds(base, _CHUNK)])


def _gather_all(uid, pid, nid, ue2, uv2, it_t):
    mesh = plsc.VectorSubcoreMesh(core_axis_name="c", subcore_axis_name="s",
                                  num_cores=2, num_subcores=16)
    urows = jax.ShapeDtypeStruct((_B, _D), jnp.float32)
    irows = jax.ShapeDtypeStruct((_B, 2 * _D), jnp.float32)
    k = pl.kernel(
        _gather_kernel_body,
        out_type=(urows, urows, irows, irows),
        mesh=mesh,
        scratch_types=[
            pltpu.VMEM((_CHUNK,), jnp.int32),
            pltpu.VMEM((_CHUNK,), jnp.int32),
            pltpu.VMEM((_CHUNK, 2 * _D), jnp.float32),
            pltpu.VMEM((_CHUNK, _D), jnp.float32),
            pltpu.VMEM((_CHUNK, _D), jnp.float32),
            pltpu.SemaphoreType.DMA,
            pltpu.SemaphoreType.DMA,
        ],
    )
    return k(uid, pid, nid, ue2, uv2, it_t)


def _loss_body(ue, uv, itp, itn, out_ref):
    i = pl.program_id(0)
    ue_ = ue[...]
    uv_ = uv[...]
    ucomb = jnp.concatenate([ue_, uv_], axis=1)
    itp_ = itp[...]
    itn_ = itn[...]
    pos = jnp.sum(ucomb * itp_, axis=1)
    neg = jnp.sum(ucomb * itn_, axis=1)
    x = pos - neg
    cf = jnp.maximum(-x, 0.0) + jnp.log1p(jnp.exp(-jnp.abs(x)))
    l2 = 0.5 * (jnp.sum(ue_ * ue_, axis=1)
                + jnp.sum(itp_[:, :_D] * itp_[:, :_D], axis=1)
                + jnp.sum(itn_[:, :_D] * itn_[:, :_D], axis=1))
    part = jnp.sum(cf + _L2_LAMBDA * l2)

    @pl.when(i == 0)
    def _():
        out_ref[0, 0] = 0.0

    out_ref[0, 0] += part

    @pl.when(i == pl.num_programs(0) - 1)
    def _():
        out_ref[0, 0] = out_ref[0, 0] / float(_B)


def _fused_loss(ue, uv, itp, itn):
    grid = _B // _LOSS_BLK
    ublk = pl.BlockSpec((_LOSS_BLK, _D), lambda i: (i, 0))
    iblk = pl.BlockSpec((_LOSS_BLK, 2 * _D), lambda i: (i, 0))
    return pl.pallas_call(
        _loss_body,
        grid=(grid,),
        in_specs=[ublk, ublk, iblk, iblk],
        out_specs=pl.BlockSpec((1, 1), lambda i: (0, 0),
                               memory_space=pltpu.SMEM),
        out_shape=jax.ShapeDtypeStruct((1, 1), jnp.float32),
    )(ue, uv, itp, itn)


def kernel(user_ids, item_pos_ids, item_neg_ids, user_embed, item_embed,
           user_visual_embed, item_visual_feature, W_vis):
    fused_items = _fuse_items(item_embed, item_visual_feature, W_vis.T)
    ue, uv, itp, itn = _gather_all(
        user_ids, item_pos_ids, item_neg_ids,
        user_embed, user_visual_embed, fused_items)
    loss = _fused_loss(ue, uv, itp, itn)
    return loss[0, 0]
